# baseline (speedup 1.0000x reference)
import functools

import jax
import jax.numpy as jnp
from jax import lax
from jax.experimental import pallas as pl
from jax.experimental.pallas import tpu as pltpu
from jax.experimental.pallas import tpu_sc as plsc

F32 = jnp.float32
I32 = jnp.int32

D_IN = 256
N_EXP = 12
N_STEP = 6
N_SUB = 5
K_TOP = 4
N_TOK = 2048
EDIMS = [2048, 2560, 3072, 3584, 4096, 2304, 2816, 3328, 3840, 4608, 3072, 4096]
SCALE = D_IN ** -0.5

BT = 256
NBLK = N_TOK // BT
S_BLOCKS = 45
S_CAP = S_BLOCKS * BT
NW = 32


def _softplus_act(v):
    return jnp.maximum(v, 0.0) + jnp.log(1.0 + jnp.exp(-jnp.abs(v)))


def _mish_act(v):
    return v * jnp.tanh(_softplus_act(v))


def _elu_act(v):
    return jnp.where(v > 0, v, jnp.exp(jnp.minimum(v, 0.0)) - 1.0)


def _selu_act(v):
    alpha = 1.6732632423543772
    scale = 1.0507009873554805
    return scale * jnp.where(v > 0, v,
                             alpha * (jnp.exp(jnp.minimum(v, 0.0)) - 1.0))


_ACTS = [jax.nn.silu, jax.nn.gelu, _mish_act, jax.nn.relu,
         _selu_act, jnp.tanh, _softplus_act, _elu_act]


def _dot(a, b):
    return jnp.dot(a, b, preferred_element_type=F32)


def _ln_k(v, g, b):
    mu = jnp.mean(v, -1, keepdims=True)
    var = jnp.mean((v - mu) ** 2, -1, keepdims=True)
    return (v - mu) / jnp.sqrt(var + 1e-5) * g + b


def _t1_body(x_ref, wT, bias, sh_upT, sh_dnT, sh_g, sh_b,
             mem_qT, mem_kT, mem_v, mem_oT, ssm_inT, ssm_outT,
             cells_upT, cells_dnT, halt_w, halt_b,
             dqT, dkT, dvT, doT, subT, morT, budT, bud_b,
             auxgT, auxg_b, col_qT, col_kT, col_vT, misc,
             acc_ref, h_ref, ti_ref, tv_ref, g2_ref, cq_ref, ck_ref, cv_ref):
    x = x_ref[...]
    base = _dot(x, wT[...]) + bias[...]
    sh = _dot(jax.nn.silu(_dot(x, sh_upT[...])), sh_dnT[...])
    sh = _ln_k(sh, sh_g[...], sh_b[...]) * misc[0, 0]
    q = _dot(x, mem_qT[...])
    att = jax.nn.softmax(_dot(q, mem_kT[...]) * SCALE, -1)
    mem = _dot(_dot(att, mem_v[...]), mem_oT[...])
    h = x + _dot(jnp.tanh(_dot(x, ssm_inT[...])), ssm_outT[...])
    bank = []
    for i in range(N_STEP):
        d = _dot(jax.nn.gelu(_dot(h, cells_upT[i])), cells_dnT[i])
        g = jax.nn.sigmoid(jnp.sum(h * halt_w[i:i + 1, :], -1, keepdims=True)
                           + halt_b[0, i])
        h = h + g * d
        bank.append(h)
    dq = _dot(h, dqT[...])
    logit_s, dv_s = [], []
    for i in range(N_STEP):
        dk_i = _dot(bank[i], dkT[...])
        dv_s.append(_dot(bank[i], dvT[...]))
        logit_s.append(jnp.sum(dq * dk_i, -1, keepdims=True))
    da = jax.nn.softmax(jnp.concatenate(logit_s, -1) * SCALE, -1)
    dctx = da[:, 0:1] * dv_s[0]
    for i in range(1, N_STEP):
        dctx = dctx + da[:, i:i + 1] * dv_s[i]
    depth = _dot(dctx, doT[...])
    gate = jax.nn.softmax(_dot(h, morT[...]), -1)
    lg = gate[:, 0:1] * _dot(h, subT[0])
    for s in range(1, N_SUB):
        lg = lg + gate[:, s:s + 1] * _dot(h, subT[s])
    probs = jax.nn.softmax(lg, -1)
    bl = _dot(h, budT[...]) + bud_b[...]
    bmax = jnp.max(bl, -1, keepdims=True)
    li5 = lax.broadcasted_iota(I32, bl.shape, 1)
    ak = jnp.min(jnp.where(bl == bmax, li5, 127), -1, keepdims=True) + 1
    li12 = lax.broadcasted_iota(I32, probs.shape, 1)
    pc = probs
    tis, tvs = [], []
    for j in range(K_TOP):
        cm = jnp.max(pc, -1, keepdims=True)
        ci = jnp.min(jnp.where(pc == cm, li12, 127), -1, keepdims=True)
        tis.append(ci)
        tvs.append(jnp.where(ak > j, cm, 0.0))
        pc = jnp.where(li12 == ci, -1.0, pc)
    tot = tvs[0] + tvs[1] + tvs[2] + tvs[3]
    den = jnp.maximum(tot, 1e-6)
    ti_ref[...] = jnp.concatenate(tis, -1)
    tv_ref[...] = jnp.concatenate(tvs, -1) / den
    g2_ref[...] = jax.nn.softmax(_dot(h, auxgT[...]) + auxg_b[...], -1)
    cq_ref[...] = _dot(h, col_qT[...])
    ck_ref[...] = _dot(h, col_kT[...])
    cv_ref[...] = _dot(h, col_vT[...])
    h_ref[...] = h
    acc_ref[...] = (base + sh * misc[0, 1] + mem * misc[0, 2]
                    + depth * misc[0, 3])


def _t2_body(x_ref, h_ref, g2_ref, aux_upT, aux_dnT, aux_g, aux_b,
             refl_upT, refl_dnT, refl_oT, ver_upT, ver_dnT,
             vqT, vkT, vvT, wg_hT, wg_vT, vg_b, corr_oT, misc,
             acc_ref, vq_ref, vk_ref, vv_ref, vg_ref):
    x = x_ref[...]
    h = h_ref[...]
    aux = _ln_k(_dot(jax.nn.gelu(_dot(x, aux_upT[...])), aux_dnT[...]),
                aux_g[...], aux_b[...])
    aux = g2_ref[:, 0:1] * aux
    rst = h
    for i in range(3):
        rst = rst + _dot(jax.nn.gelu(_dot(rst, refl_upT[i])), refl_dnT[i])
    refl = _dot(rst, refl_oT[...])
    v = h
    for i in range(2):
        v = v + _dot(jax.nn.gelu(_dot(v, ver_upT[i])), ver_dnT[i])
    vg = jax.nn.softmax(_dot(h, wg_hT[...]) + _dot(v, wg_vT[...]) + vg_b[...],
                        -1)
    corr = vg[:, 1:2] * _dot(v, corr_oT[...])
    acc_ref[...] = misc[0, 4] * (aux + refl) + misc[0, 5] * corr
    vq_ref[...] = _dot(v, vqT[...])
    vk_ref[...] = _dot(v, vkT[...])
    vv_ref[...] = _dot(v, vvT[...])
    vg_ref[...] = vg


def _t3_body(q_ref, k_ref, v_ref, woT_ref, gate_ref, misc_ref, out_ref):
    qb = q_ref[...]
    sc = lax.dot_general(qb, k_ref[...], (((1,), (1,)), ((), ())),
                         preferred_element_type=F32)
    att = jax.nn.softmax(sc * SCALE, -1)
    ctx = _dot(att, v_ref[...])
    out_ref[...] = misc_ref[0, 0] * gate_ref[:, 0:1] * _dot(ctx, woT_ref[...])


def _attention(q, k, v, woT, gate, mscale):
    misc = jnp.reshape(jnp.stack([mscale] * 8), (1, 8)).astype(F32)
    full = lambda i: (0, 0)
    return pl.pallas_call(
        _t3_body,
        grid=(NBLK,),
        in_specs=[
            pl.BlockSpec((BT, D_IN), lambda i: (i, 0)),
            pl.BlockSpec((N_TOK, D_IN), full),
            pl.BlockSpec((N_TOK, D_IN), full),
            pl.BlockSpec((D_IN, D_IN), full),
            pl.BlockSpec((BT, 2), lambda i: (i, 0)),
            pl.BlockSpec((1, 8), full),
        ],
        out_specs=pl.BlockSpec((BT, D_IN), lambda i: (i, 0)),
        out_shape=jax.ShapeDtypeStruct((N_TOK, D_IN), F32),
    )(q, k, v, woT, gate, misc)


def _expert_body(act, s_ref, xs_ref, upT_ref, dnT_ref, g_ref, b_ref,
                 wpk_ref, ysin_ref, ys_ref):
    i = pl.program_id(0)

    @pl.when(i < s_ref[1])
    def _():
        y = act(_dot(xs_ref[...], upT_ref[...]))
        o = _dot(y, dnT_ref[...])
        o = _ln_k(o, g_ref[...], b_ref[...])
        ys_ref[...] = o * wpk_ref[...]


def _expert_call(e, sarr, xs, upT, dnT, g2d, b2d, wpk, ys):
    dim = EDIMS[e]

    def clamp(i, s):
        return (s[0] + jnp.minimum(i, jnp.maximum(s[1] - 1, 0)), 0)

    full = lambda i, s: (0, 0)
    gs = pltpu.PrefetchScalarGridSpec(
        num_scalar_prefetch=1,
        grid=(NBLK,),
        in_specs=[
            pl.BlockSpec((BT, D_IN), clamp),
            pl.BlockSpec((D_IN, dim), full),
            pl.BlockSpec((dim, D_IN), full),
            pl.BlockSpec((1, D_IN), full),
            pl.BlockSpec((1, D_IN), full),
            pl.BlockSpec((BT, 1), clamp),
            pl.BlockSpec((BT, D_IN), clamp),
        ],
        out_specs=pl.BlockSpec((BT, D_IN), clamp),
    )
    return pl.pallas_call(
        functools.partial(_expert_body, _ACTS[e % 8]),
        grid_spec=gs,
        out_shape=jax.ShapeDtypeStruct((S_CAP, D_IN), F32),
        input_output_aliases={7: 0},
    )(sarr, xs, upT, dnT, g2d, b2d, wpk, ys)


def _sc_mesh():
    return plsc.VectorSubcoreMesh(core_axis_name="c", subcore_axis_name="s")


def _sc_gather_rows(table, idx, n_rows):
    rpw = n_rows // NW
    nch = 9 if rpw % 9 == 0 else 8
    cpr = rpw // nch

    @functools.partial(
        pl.kernel, mesh=_sc_mesh(),
        out_type=jax.ShapeDtypeStruct((n_rows, D_IN), F32),
        scratch_types=[
            pltpu.VMEM((rpw,), I32),
            pltpu.VMEM((rpw, D_IN), F32),
            pltpu.SemaphoreType.DMA,
            pltpu.SemaphoreType.DMA,
        ],
    )
    def gk(table_hbm, idx_hbm, out_hbm, idx_v, rows_v, gsem, wsem):
        wid = lax.axis_index("s") * 2 + lax.axis_index("c")
        b0 = wid * rpw
        pltpu.sync_copy(idx_hbm.at[pl.ds(b0, rpw)], idx_v)
        gets = [
            pltpu.async_copy(
                table_hbm.at[idx_v.at[pl.ds(c * cpr, cpr)]],
                rows_v.at[pl.ds(c * cpr, cpr)], gsem)
            for c in range(nch)
        ]
        puts = []
        for c in range(nch):
            gets[c].wait()
            puts.append(pltpu.async_copy(
                rows_v.at[pl.ds(c * cpr, cpr)],
                out_hbm.at[pl.ds(b0 + c * cpr, cpr)], wsem))
        for c in range(nch):
            puts[c].wait()

    return gk(table, idx)


def _sc_combine(ys, pos_flat):
    tpw = N_TOK // NW
    spw = tpw * K_TOP

    @functools.partial(
        pl.kernel, mesh=_sc_mesh(),
        out_type=jax.ShapeDtypeStruct((N_TOK, D_IN), F32),
        scratch_types=[
            pltpu.VMEM((spw,), I32),
            pltpu.VMEM((spw, D_IN), F32),
            pltpu.VMEM((tpw, D_IN), F32),
            pltpu.SemaphoreType.DMA,
        ],
    )
    def ck(ys_hbm, pos_hbm, out_hbm, idx_v, rows_v, acc_v, sem):
        wid = lax.axis_index("s") * 2 + lax.axis_index("c")
        tb = wid * tpw
        pltpu.sync_copy(pos_hbm.at[pl.ds(tb * K_TOP, spw)], idx_v)
        nch, cpr = 8, spw // 8
        gets = [
            pltpu.async_copy(
                ys_hbm.at[idx_v.at[pl.ds(c * cpr, cpr)]],
                rows_v.at[pl.ds(c * cpr, cpr)], sem)
            for c in range(nch)
        ]
        for c in range(nch):
            gets[c].wait()

        def tok(t, carry):
            for c in range(D_IN // 16):
                sl = pl.ds(c * 16, 16)
                acc = (rows_v[4 * t + 0, sl] + rows_v[4 * t + 1, sl]
                       + rows_v[4 * t + 2, sl] + rows_v[4 * t + 3, sl])
                acc_v[t, sl] = acc
            return carry

        lax.fori_loop(0, tpw, tok, 0)
        pltpu.sync_copy(acc_v, out_hbm.at[pl.ds(tb, tpw)])

    return ck(ys, pos_flat)


def kernel(x, params):
    p = params
    full = lambda i: (0, 0)
    full3 = lambda i: (0, 0, 0)
    blk = lambda i: (i, 0)

    m = jnp.tanh(p['mix'].astype(F32))
    misc = jnp.reshape(
        jnp.stack([p['shared_scale'].astype(F32), m[0], m[1], m[2], m[4],
                   m[5], jnp.zeros(()), jnp.zeros(())]), (1, 8))

    wT = p['weight'].T
    bias = p['bias'].reshape(1, -1)
    sh_upT = p['shared_up'].T
    sh_dnT = p['shared_down'].T
    sh_g = p['shared_g'].reshape(1, -1)
    sh_b = p['shared_b'].reshape(1, -1)
    mem_qT = p['mem_q'].T
    mem_kT = p['mem_k'].T
    mem_oT = p['mem_o'].T
    ssm_inT = jnp.pad(p['ssm_in'].T, ((0, 0), (0, 80)))
    ssm_outT = jnp.pad(p['ssm_out'].T, ((0, 80), (0, 0)))
    cells_upT = jnp.transpose(p['cells_up'], (0, 2, 1))
    cells_dnT = jnp.transpose(p['cells_down'], (0, 2, 1))
    halt_w = p['halt_w'].reshape(N_STEP, D_IN)
    halt_b = p['halt_b'].reshape(1, N_STEP)
    dqT, dkT = p['depth_q'].T, p['depth_k'].T
    dvT, doT = p['depth_v'].T, p['depth_o'].T
    subT = jnp.transpose(p['sub_routers'], (0, 2, 1))
    morT = p['mor_gate'].T
    budT = p['budget_w'].T
    bud_b = p['budget_b'].reshape(1, -1)
    auxgT = p['aux_gate_w'].T
    auxg_b = p['aux_gate_b'].reshape(1, -1)
    col_qT, col_kT, col_vT = p['col_q'].T, p['col_k'].T, p['col_v'].T
    col_oT = p['col_o'].T

    t1_outs = pl.pallas_call(
        _t1_body,
        grid=(NBLK,),
        in_specs=[
            pl.BlockSpec((BT, D_IN), blk),
            pl.BlockSpec((D_IN, D_IN), full), pl.BlockSpec((1, D_IN), full),
            pl.BlockSpec((D_IN, 3072), full), pl.BlockSpec((3072, D_IN), full),
            pl.BlockSpec((1, D_IN), full), pl.BlockSpec((1, D_IN), full),
            pl.BlockSpec((D_IN, D_IN), full), pl.BlockSpec((D_IN, 32), full),
            pl.BlockSpec((32, D_IN), full), pl.BlockSpec((D_IN, D_IN), full),
            pl.BlockSpec((D_IN, 128), full), pl.BlockSpec((128, D_IN), full),
            pl.BlockSpec((N_STEP, D_IN, 896), full3),
            pl.BlockSpec((N_STEP, 896, D_IN), full3),
            pl.BlockSpec((N_STEP, D_IN), full), pl.BlockSpec((1, N_STEP), full),
            pl.BlockSpec((D_IN, D_IN), full), pl.BlockSpec((D_IN, D_IN), full),
            pl.BlockSpec((D_IN, D_IN), full), pl.BlockSpec((D_IN, D_IN), full),
            pl.BlockSpec((N_SUB, D_IN, N_EXP), full3),
            pl.BlockSpec((D_IN, N_SUB), full),
            pl.BlockSpec((D_IN, N_SUB), full), pl.BlockSpec((1, N_SUB), full),
            pl.BlockSpec((D_IN, 2), full), pl.BlockSpec((1, 2), full),
            pl.BlockSpec((D_IN, D_IN), full), pl.BlockSpec((D_IN, D_IN), full),
            pl.BlockSpec((D_IN, D_IN), full), pl.BlockSpec((1, 8), full),
        ],
        out_specs=[
            pl.BlockSpec((BT, D_IN), blk), pl.BlockSpec((BT, D_IN), blk),
            pl.BlockSpec((BT, K_TOP), blk), pl.BlockSpec((BT, K_TOP), blk),
            pl.BlockSpec((BT, 2), blk), pl.BlockSpec((BT, D_IN), blk),
            pl.BlockSpec((BT, D_IN), blk), pl.BlockSpec((BT, D_IN), blk),
        ],
        out_shape=[
            jax.ShapeDtypeStruct((N_TOK, D_IN), F32),
            jax.ShapeDtypeStruct((N_TOK, D_IN), F32),
            jax.ShapeDtypeStruct((N_TOK, K_TOP), I32),
            jax.ShapeDtypeStruct((N_TOK, K_TOP), F32),
            jax.ShapeDtypeStruct((N_TOK, 2), F32),
            jax.ShapeDtypeStruct((N_TOK, D_IN), F32),
            jax.ShapeDtypeStruct((N_TOK, D_IN), F32),
            jax.ShapeDtypeStruct((N_TOK, D_IN), F32),
        ],
    )(x, wT, bias, sh_upT, sh_dnT, sh_g, sh_b,
      mem_qT, mem_kT, p['mem_v'], mem_oT, ssm_inT, ssm_outT,
      cells_upT, cells_dnT, halt_w, halt_b,
      dqT, dkT, dvT, doT, subT, morT, budT, bud_b,
      auxgT, auxg_b, col_qT, col_kT, col_vT, misc)
    acc1, h, ti, tv, g2, cq, ck, cv = t1_outs

    aux_upT = p['aux_up'].T
    aux_dnT = p['aux_down'].T
    aux_g = p['aux_g'].reshape(1, -1)
    aux_b = p['aux_b'].reshape(1, -1)
    refl_upT = jnp.transpose(p['refl_up'], (0, 2, 1))
    refl_dnT = jnp.transpose(p['refl_down'], (0, 2, 1))
    refl_oT = p['refl_o'].T
    ver_upT = jnp.transpose(p['ver_up'], (0, 2, 1))
    ver_dnT = jnp.transpose(p['ver_down'], (0, 2, 1))
    vqT, vkT, vvT = p['ver_q'].T, p['ver_k'].T, p['ver_v'].T
    ver_oT = p['ver_o'].T
    wg_hT = p['ver_gate_w'][:, :D_IN].T
    wg_vT = p['ver_gate_w'][:, D_IN:].T
    vg_b = p['ver_gate_b'].reshape(1, -1)
    corr_oT = p['corr_o'].T

    t2_outs = pl.pallas_call(
        _t2_body,
        grid=(NBLK,),
        in_specs=[
            pl.BlockSpec((BT, D_IN), blk), pl.BlockSpec((BT, D_IN), blk),
            pl.BlockSpec((BT, 2), blk),
            pl.BlockSpec((D_IN, 4096), full), pl.BlockSpec((4096, D_IN), full),
            pl.BlockSpec((1, D_IN), full), pl.BlockSpec((1, D_IN), full),
            pl.BlockSpec((3, D_IN, 1280), full3),
            pl.BlockSpec((3, 1280, D_IN), full3),
            pl.BlockSpec((D_IN, D_IN), full),
            pl.BlockSpec((2, D_IN, 1536), full3),
            pl.BlockSpec((2, 1536, D_IN), full3),
            pl.BlockSpec((D_IN, D_IN), full), pl.BlockSpec((D_IN, D_IN), full),
            pl.BlockSpec((D_IN, D_IN), full),
            pl.BlockSpec((D_IN, 2), full), pl.BlockSpec((D_IN, 2), full),
            pl.BlockSpec((1, 2), full),
            pl.BlockSpec((D_IN, D_IN), full), pl.BlockSpec((1, 8), full),
        ],
        out_specs=[
            pl.BlockSpec((BT, D_IN), blk), pl.BlockSpec((BT, D_IN), blk),
            pl.BlockSpec((BT, D_IN), blk), pl.BlockSpec((BT, D_IN), blk),
            pl.BlockSpec((BT, 2), blk),
        ],
        out_shape=[
            jax.ShapeDtypeStruct((N_TOK, D_IN), F32),
            jax.ShapeDtypeStruct((N_TOK, D_IN), F32),
            jax.ShapeDtypeStruct((N_TOK, D_IN), F32),
            jax.ShapeDtypeStruct((N_TOK, D_IN), F32),
            jax.ShapeDtypeStruct((N_TOK, 2), F32),
        ],
    )(x, h, g2, aux_upT, aux_dnT, aux_g, aux_b,
      refl_upT, refl_dnT, refl_oT, ver_upT, ver_dnT,
      vqT, vkT, vvT, wg_hT, wg_vT, vg_b, corr_oT, misc)
    acc2, vq, vk, vv, vg = t2_outs

    ones2 = jnp.ones((N_TOK, 2), F32)
    col_c = _attention(cq, ck, cv, col_oT, ones2, m[5])
    ver_c = _attention(vq, vk, vv, ver_oT, vg, m[5])

    flat_e = jnp.where(tv > 0, ti, N_EXP).reshape(-1).astype(I32)
    perm = jnp.argsort(flat_e, stable=True).astype(I32)
    sorted_e = flat_e[perm]
    counts13 = jnp.bincount(flat_e, length=N_EXP + 1).astype(I32)
    nb = (counts13[:N_EXP] + BT - 1) // BT
    bo = jnp.concatenate([jnp.zeros(1, I32), jnp.cumsum(nb)])[:N_EXP]
    gs0 = jnp.concatenate([jnp.zeros(1, I32),
                           jnp.cumsum(counts13)])[:N_EXP + 1]
    rank = jnp.arange(flat_e.shape[0], dtype=I32) - gs0[sorted_e]
    packed = jnp.where(sorted_e < N_EXP, bo[sorted_e] * BT + rank,
                       S_CAP - 1).astype(I32)
    tok_packed = jnp.zeros((S_CAP,), I32).at[packed].set(perm // K_TOP)
    w_perm = tv.reshape(-1)[perm] * m[3]
    w_packed = jnp.zeros((S_CAP,), F32).at[packed].set(w_perm)
    w_packed = w_packed.reshape(S_CAP, 1)
    pos_flat = jnp.zeros((flat_e.shape[0],), I32).at[perm].set(packed)

    xs = _sc_gather_rows(x, tok_packed, S_CAP)
    ys = jnp.zeros((S_CAP, D_IN), F32)
    for e in range(N_EXP):
        sarr = jnp.stack([bo[e], nb[e]]).astype(I32)
        upT = p['experts_up'][e].T
        dnT = p['experts_down'][e].T
        g2d = p['exp_g'][e].reshape(1, -1)
        b2d = p['exp_b'][e].reshape(1, -1)
        ys = _expert_call(e, sarr, xs, upT, dnT, g2d, b2d, w_packed, ys)
    moe = _sc_combine(ys, pos_flat)

    return acc1 + acc2 + col_c + ver_c + moe


# speedup vs baseline: 1.1329x; 1.1329x over previous
import functools

import jax
import jax.numpy as jnp
from jax import lax
from jax.experimental import pallas as pl
from jax.experimental.pallas import tpu as pltpu
from jax.experimental.pallas import tpu_sc as plsc

F32 = jnp.float32
I32 = jnp.int32

D_IN = 256
N_EXP = 12
N_STEP = 6
N_SUB = 5
K_TOP = 4
N_TOK = 2048
EDIMS = [2048, 2560, 3072, 3584, 4096, 2304, 2816, 3328, 3840, 4608, 3072, 4096]
SCALE = D_IN ** -0.5

BT = 256
NBLK = N_TOK // BT
S_BLOCKS = 45
S_CAP = S_BLOCKS * BT
NW = 32


def _softplus_act(v):
    return jnp.maximum(v, 0.0) + jnp.log(1.0 + jnp.exp(-jnp.abs(v)))


def _mish_act(v):
    return v * jnp.tanh(_softplus_act(v))


def _elu_act(v):
    return jnp.where(v > 0, v, jnp.exp(jnp.minimum(v, 0.0)) - 1.0)


def _selu_act(v):
    alpha = 1.6732632423543772
    scale = 1.0507009873554805
    return scale * jnp.where(v > 0, v,
                             alpha * (jnp.exp(jnp.minimum(v, 0.0)) - 1.0))


_ACTS = [jax.nn.silu, jax.nn.gelu, _mish_act, jax.nn.relu,
         _selu_act, jnp.tanh, _softplus_act, _elu_act]


def _dot(a, b):
    return jnp.dot(a, b, preferred_element_type=F32)


def _ln_k(v, g, b):
    mu = jnp.mean(v, -1, keepdims=True)
    var = jnp.mean((v - mu) ** 2, -1, keepdims=True)
    return (v - mu) / jnp.sqrt(var + 1e-5) * g + b


def _t1_body(x_ref, wT, bias, sh_upT, sh_dnT, sh_g, sh_b,
             mem_qT, mem_kT, mem_v, mem_oT, ssm_inT, ssm_outT,
             cells_upT, cells_dnT, halt_w, halt_b,
             dqT, dkT, dvT, doT, subT, morT, budT, bud_b,
             auxgT, auxg_b, col_qT, col_kT, col_vT, misc,
             acc_ref, h_ref, ti_ref, tv_ref, g2_ref, cq_ref, ck_ref, cv_ref):
    x = x_ref[...]
    base = _dot(x, wT[...]) + bias[...]
    sh = _dot(jax.nn.silu(_dot(x, sh_upT[...])), sh_dnT[...])
    sh = _ln_k(sh, sh_g[...], sh_b[...]) * misc[0, 0]
    q = _dot(x, mem_qT[...])
    att = jax.nn.softmax(_dot(q, mem_kT[...]) * SCALE, -1)
    mem = _dot(_dot(att, mem_v[...]), mem_oT[...])
    h = x + _dot(jnp.tanh(_dot(x, ssm_inT[...])), ssm_outT[...])
    bank = []
    for i in range(N_STEP):
        d = _dot(jax.nn.gelu(_dot(h, cells_upT[i])), cells_dnT[i])
        g = jax.nn.sigmoid(jnp.sum(h * halt_w[i:i + 1, :], -1, keepdims=True)
                           + halt_b[0, i])
        h = h + g * d
        bank.append(h)
    dq = _dot(h, dqT[...])
    logit_s, dv_s = [], []
    for i in range(N_STEP):
        dk_i = _dot(bank[i], dkT[...])
        dv_s.append(_dot(bank[i], dvT[...]))
        logit_s.append(jnp.sum(dq * dk_i, -1, keepdims=True))
    da = jax.nn.softmax(jnp.concatenate(logit_s, -1) * SCALE, -1)
    dctx = da[:, 0:1] * dv_s[0]
    for i in range(1, N_STEP):
        dctx = dctx + da[:, i:i + 1] * dv_s[i]
    depth = _dot(dctx, doT[...])
    gate = jax.nn.softmax(_dot(h, morT[...]), -1)
    lg = gate[:, 0:1] * _dot(h, subT[0])
    for s in range(1, N_SUB):
        lg = lg + gate[:, s:s + 1] * _dot(h, subT[s])
    probs = jax.nn.softmax(lg, -1)
    bl = _dot(h, budT[...]) + bud_b[...]
    bmax = jnp.max(bl, -1, keepdims=True)
    li5 = lax.broadcasted_iota(I32, bl.shape, 1)
    ak = jnp.min(jnp.where(bl == bmax, li5, 127), -1, keepdims=True) + 1
    li12 = lax.broadcasted_iota(I32, probs.shape, 1)
    pc = probs
    tis, tvs = [], []
    for j in range(K_TOP):
        cm = jnp.max(pc, -1, keepdims=True)
        ci = jnp.min(jnp.where(pc == cm, li12, 127), -1, keepdims=True)
        tis.append(ci)
        tvs.append(jnp.where(ak > j, cm, 0.0))
        pc = jnp.where(li12 == ci, -1.0, pc)
    tot = tvs[0] + tvs[1] + tvs[2] + tvs[3]
    den = jnp.maximum(tot, 1e-6)
    ti_ref[...] = jnp.concatenate(tis, -1)
    tv_ref[...] = jnp.concatenate(tvs, -1) / den
    g2_ref[...] = jax.nn.softmax(_dot(h, auxgT[...]) + auxg_b[...], -1)
    cq_ref[...] = _dot(h, col_qT[...])
    ck_ref[...] = _dot(h, col_kT[...])
    cv_ref[...] = _dot(h, col_vT[...])
    h_ref[...] = h
    acc_ref[...] = (base + sh * misc[0, 1] + mem * misc[0, 2]
                    + depth * misc[0, 3])


def _t2_body(x_ref, h_ref, g2_ref, aux_upT, aux_dnT, aux_g, aux_b,
             refl_upT, refl_dnT, refl_oT, ver_upT, ver_dnT,
             vqT, vkT, vvT, wg_hT, wg_vT, vg_b, corr_oT, misc,
             acc_ref, vq_ref, vk_ref, vv_ref, vg_ref):
    x = x_ref[...]
    h = h_ref[...]
    aux = _ln_k(_dot(jax.nn.gelu(_dot(x, aux_upT[...])), aux_dnT[...]),
                aux_g[...], aux_b[...])
    aux = g2_ref[:, 0:1] * aux
    rst = h
    for i in range(3):
        rst = rst + _dot(jax.nn.gelu(_dot(rst, refl_upT[i])), refl_dnT[i])
    refl = _dot(rst, refl_oT[...])
    v = h
    for i in range(2):
        v = v + _dot(jax.nn.gelu(_dot(v, ver_upT[i])), ver_dnT[i])
    vg = jax.nn.softmax(_dot(h, wg_hT[...]) + _dot(v, wg_vT[...]) + vg_b[...],
                        -1)
    corr = vg[:, 1:2] * _dot(v, corr_oT[...])
    acc_ref[...] = misc[0, 4] * (aux + refl) + misc[0, 5] * corr
    vq_ref[...] = _dot(v, vqT[...])
    vk_ref[...] = _dot(v, vkT[...])
    vv_ref[...] = _dot(v, vvT[...])
    vg_ref[...] = vg


def _t3_body(q_ref, k_ref, v_ref, woT_ref, gate_ref, misc_ref, out_ref):
    qb = q_ref[...]
    sc = lax.dot_general(qb, k_ref[...], (((1,), (1,)), ((), ())),
                         preferred_element_type=F32)
    att = jax.nn.softmax(sc * SCALE, -1)
    ctx = _dot(att, v_ref[...])
    out_ref[...] = misc_ref[0, 0] * gate_ref[:, 0:1] * _dot(ctx, woT_ref[...])


def _attention(q, k, v, woT, gate, mscale):
    misc = jnp.reshape(jnp.stack([mscale] * 8), (1, 8)).astype(F32)
    full = lambda i: (0, 0)
    return pl.pallas_call(
        _t3_body,
        grid=(NBLK,),
        in_specs=[
            pl.BlockSpec((BT, D_IN), lambda i: (i, 0)),
            pl.BlockSpec((N_TOK, D_IN), full),
            pl.BlockSpec((N_TOK, D_IN), full),
            pl.BlockSpec((D_IN, D_IN), full),
            pl.BlockSpec((BT, 2), lambda i: (i, 0)),
            pl.BlockSpec((1, 8), full),
        ],
        out_specs=pl.BlockSpec((BT, D_IN), lambda i: (i, 0)),
        out_shape=jax.ShapeDtypeStruct((N_TOK, D_IN), F32),
    )(q, k, v, woT, gate, misc)


def _expert_body(act, s_ref, x_ref, idx_ref, upT_ref, dnT_ref, g_ref, b_ref,
                 wpk_ref, ysin_ref, ys_ref):
    i = pl.program_id(0)

    @pl.when(i < s_ref[1])
    def _():
        oh = (lax.broadcasted_iota(I32, (BT, N_TOK), 1)
              == idx_ref[...]).astype(F32)
        xb = _dot(oh, x_ref[...])
        y = act(_dot(xb, upT_ref[...]))
        o = _dot(y, dnT_ref[...])
        o = _ln_k(o, g_ref[...], b_ref[...])
        ys_ref[...] = o * wpk_ref[...]


def _expert_call(e, sarr, x, tok_packed, upT, dnT, g2d, b2d, wpk, ys):
    dim = EDIMS[e]

    def clamp(i, s):
        return (s[0] + jnp.minimum(i, jnp.maximum(s[1] - 1, 0)), 0)

    full = lambda i, s: (0, 0)
    gs = pltpu.PrefetchScalarGridSpec(
        num_scalar_prefetch=1,
        grid=(NBLK,),
        in_specs=[
            pl.BlockSpec((N_TOK, D_IN), full),
            pl.BlockSpec((BT, 1), clamp),
            pl.BlockSpec((D_IN, dim), full),
            pl.BlockSpec((dim, D_IN), full),
            pl.BlockSpec((1, D_IN), full),
            pl.BlockSpec((1, D_IN), full),
            pl.BlockSpec((BT, 1), clamp),
            pl.BlockSpec((BT, D_IN), clamp),
        ],
        out_specs=pl.BlockSpec((BT, D_IN), clamp),
    )
    return pl.pallas_call(
        functools.partial(_expert_body, _ACTS[e % 8]),
        grid_spec=gs,
        out_shape=jax.ShapeDtypeStruct((S_CAP, D_IN), F32),
        input_output_aliases={8: 0},
    )(sarr, x, tok_packed, upT, dnT, g2d, b2d, wpk, ys)


def _sc_mesh():
    return plsc.VectorSubcoreMesh(core_axis_name="c", subcore_axis_name="s")


def _sc_gather_rows(table, idx, n_rows):
    rpw = n_rows // NW
    nch = 9 if rpw % 9 == 0 else 8
    cpr = rpw // nch

    @functools.partial(
        pl.kernel, mesh=_sc_mesh(),
        out_type=jax.ShapeDtypeStruct((n_rows, D_IN), F32),
        scratch_types=[
            pltpu.VMEM((rpw,), I32),
            pltpu.VMEM((rpw, D_IN), F32),
            pltpu.SemaphoreType.DMA,
            pltpu.SemaphoreType.DMA,
        ],
    )
    def gk(table_hbm, idx_hbm, out_hbm, idx_v, rows_v, gsem, wsem):
        wid = lax.axis_index("s") * 2 + lax.axis_index("c")
        b0 = wid * rpw
        pltpu.sync_copy(idx_hbm.at[pl.ds(b0, rpw)], idx_v)
        gets = [
            pltpu.async_copy(
                table_hbm.at[idx_v.at[pl.ds(c * cpr, cpr)]],
                rows_v.at[pl.ds(c * cpr, cpr)], gsem)
            for c in range(nch)
        ]
        puts = []
        for c in range(nch):
            gets[c].wait()
            puts.append(pltpu.async_copy(
                rows_v.at[pl.ds(c * cpr, cpr)],
                out_hbm.at[pl.ds(b0 + c * cpr, cpr)], wsem))
        for c in range(nch):
            puts[c].wait()

    return gk(table, idx)


def _sc_combine(ys, pos_flat):
    tpw = N_TOK // NW
    spw = tpw * K_TOP

    @functools.partial(
        pl.kernel, mesh=_sc_mesh(),
        out_type=jax.ShapeDtypeStruct((N_TOK, D_IN), F32),
        scratch_types=[
            pltpu.VMEM((spw,), I32),
            pltpu.VMEM((spw, D_IN), F32),
            pltpu.VMEM((tpw, D_IN), F32),
            pltpu.SemaphoreType.DMA,
        ],
    )
    def ck(ys_hbm, pos_hbm, out_hbm, idx_v, rows_v, acc_v, sem):
        wid = lax.axis_index("s") * 2 + lax.axis_index("c")
        tb = wid * tpw
        pltpu.sync_copy(pos_hbm.at[pl.ds(tb * K_TOP, spw)], idx_v)
        nch, cpr = 8, spw // 8
        gets = [
            pltpu.async_copy(
                ys_hbm.at[idx_v.at[pl.ds(c * cpr, cpr)]],
                rows_v.at[pl.ds(c * cpr, cpr)], sem)
            for c in range(nch)
        ]
        for c in range(nch):
            gets[c].wait()

        def tok(t, carry):
            for c in range(D_IN // 16):
                sl = pl.ds(c * 16, 16)
                acc = (rows_v[4 * t + 0, sl] + rows_v[4 * t + 1, sl]
                       + rows_v[4 * t + 2, sl] + rows_v[4 * t + 3, sl])
                acc_v[t, sl] = acc
            return carry

        lax.fori_loop(0, tpw, tok, 0)
        pltpu.sync_copy(acc_v, out_hbm.at[pl.ds(tb, tpw)])

    return ck(ys, pos_flat)


def kernel(x, params):
    p = params
    full = lambda i: (0, 0)
    full3 = lambda i: (0, 0, 0)
    blk = lambda i: (i, 0)

    m = jnp.tanh(p['mix'].astype(F32))
    misc = jnp.reshape(
        jnp.stack([p['shared_scale'].astype(F32), m[0], m[1], m[2], m[4],
                   m[5], jnp.zeros(()), jnp.zeros(())]), (1, 8))

    wT = p['weight'].T
    bias = p['bias'].reshape(1, -1)
    sh_upT = p['shared_up'].T
    sh_dnT = p['shared_down'].T
    sh_g = p['shared_g'].reshape(1, -1)
    sh_b = p['shared_b'].reshape(1, -1)
    mem_qT = p['mem_q'].T
    mem_kT = p['mem_k'].T
    mem_oT = p['mem_o'].T
    ssm_inT = jnp.pad(p['ssm_in'].T, ((0, 0), (0, 80)))
    ssm_outT = jnp.pad(p['ssm_out'].T, ((0, 80), (0, 0)))
    cells_upT = jnp.transpose(p['cells_up'], (0, 2, 1))
    cells_dnT = jnp.transpose(p['cells_down'], (0, 2, 1))
    halt_w = p['halt_w'].reshape(N_STEP, D_IN)
    halt_b = p['halt_b'].reshape(1, N_STEP)
    dqT, dkT = p['depth_q'].T, p['depth_k'].T
    dvT, doT = p['depth_v'].T, p['depth_o'].T
    subT = jnp.transpose(p['sub_routers'], (0, 2, 1))
    morT = p['mor_gate'].T
    budT = p['budget_w'].T
    bud_b = p['budget_b'].reshape(1, -1)
    auxgT = p['aux_gate_w'].T
    auxg_b = p['aux_gate_b'].reshape(1, -1)
    col_qT, col_kT, col_vT = p['col_q'].T, p['col_k'].T, p['col_v'].T
    col_oT = p['col_o'].T

    t1_outs = pl.pallas_call(
        _t1_body,
        grid=(NBLK,),
        in_specs=[
            pl.BlockSpec((BT, D_IN), blk),
            pl.BlockSpec((D_IN, D_IN), full), pl.BlockSpec((1, D_IN), full),
            pl.BlockSpec((D_IN, 3072), full), pl.BlockSpec((3072, D_IN), full),
            pl.BlockSpec((1, D_IN), full), pl.BlockSpec((1, D_IN), full),
            pl.BlockSpec((D_IN, D_IN), full), pl.BlockSpec((D_IN, 32), full),
            pl.BlockSpec((32, D_IN), full), pl.BlockSpec((D_IN, D_IN), full),
            pl.BlockSpec((D_IN, 128), full), pl.BlockSpec((128, D_IN), full),
            pl.BlockSpec((N_STEP, D_IN, 896), full3),
            pl.BlockSpec((N_STEP, 896, D_IN), full3),
            pl.BlockSpec((N_STEP, D_IN), full), pl.BlockSpec((1, N_STEP), full),
            pl.BlockSpec((D_IN, D_IN), full), pl.BlockSpec((D_IN, D_IN), full),
            pl.BlockSpec((D_IN, D_IN), full), pl.BlockSpec((D_IN, D_IN), full),
            pl.BlockSpec((N_SUB, D_IN, N_EXP), full3),
            pl.BlockSpec((D_IN, N_SUB), full),
            pl.BlockSpec((D_IN, N_SUB), full), pl.BlockSpec((1, N_SUB), full),
            pl.BlockSpec((D_IN, 2), full), pl.BlockSpec((1, 2), full),
            pl.BlockSpec((D_IN, D_IN), full), pl.BlockSpec((D_IN, D_IN), full),
            pl.BlockSpec((D_IN, D_IN), full), pl.BlockSpec((1, 8), full),
        ],
        out_specs=[
            pl.BlockSpec((BT, D_IN), blk), pl.BlockSpec((BT, D_IN), blk),
            pl.BlockSpec((BT, K_TOP), blk), pl.BlockSpec((BT, K_TOP), blk),
            pl.BlockSpec((BT, 2), blk), pl.BlockSpec((BT, D_IN), blk),
            pl.BlockSpec((BT, D_IN), blk), pl.BlockSpec((BT, D_IN), blk),
        ],
        out_shape=[
            jax.ShapeDtypeStruct((N_TOK, D_IN), F32),
            jax.ShapeDtypeStruct((N_TOK, D_IN), F32),
            jax.ShapeDtypeStruct((N_TOK, K_TOP), I32),
            jax.ShapeDtypeStruct((N_TOK, K_TOP), F32),
            jax.ShapeDtypeStruct((N_TOK, 2), F32),
            jax.ShapeDtypeStruct((N_TOK, D_IN), F32),
            jax.ShapeDtypeStruct((N_TOK, D_IN), F32),
            jax.ShapeDtypeStruct((N_TOK, D_IN), F32),
        ],
    )(x, wT, bias, sh_upT, sh_dnT, sh_g, sh_b,
      mem_qT, mem_kT, p['mem_v'], mem_oT, ssm_inT, ssm_outT,
      cells_upT, cells_dnT, halt_w, halt_b,
      dqT, dkT, dvT, doT, subT, morT, budT, bud_b,
      auxgT, auxg_b, col_qT, col_kT, col_vT, misc)
    acc1, h, ti, tv, g2, cq, ck, cv = t1_outs

    aux_upT = p['aux_up'].T
    aux_dnT = p['aux_down'].T
    aux_g = p['aux_g'].reshape(1, -1)
    aux_b = p['aux_b'].reshape(1, -1)
    refl_upT = jnp.transpose(p['refl_up'], (0, 2, 1))
    refl_dnT = jnp.transpose(p['refl_down'], (0, 2, 1))
    refl_oT = p['refl_o'].T
    ver_upT = jnp.transpose(p['ver_up'], (0, 2, 1))
    ver_dnT = jnp.transpose(p['ver_down'], (0, 2, 1))
    vqT, vkT, vvT = p['ver_q'].T, p['ver_k'].T, p['ver_v'].T
    ver_oT = p['ver_o'].T
    wg_hT = p['ver_gate_w'][:, :D_IN].T
    wg_vT = p['ver_gate_w'][:, D_IN:].T
    vg_b = p['ver_gate_b'].reshape(1, -1)
    corr_oT = p['corr_o'].T

    t2_outs = pl.pallas_call(
        _t2_body,
        grid=(NBLK,),
        in_specs=[
            pl.BlockSpec((BT, D_IN), blk), pl.BlockSpec((BT, D_IN), blk),
            pl.BlockSpec((BT, 2), blk),
            pl.BlockSpec((D_IN, 4096), full), pl.BlockSpec((4096, D_IN), full),
            pl.BlockSpec((1, D_IN), full), pl.BlockSpec((1, D_IN), full),
            pl.BlockSpec((3, D_IN, 1280), full3),
            pl.BlockSpec((3, 1280, D_IN), full3),
            pl.BlockSpec((D_IN, D_IN), full),
            pl.BlockSpec((2, D_IN, 1536), full3),
            pl.BlockSpec((2, 1536, D_IN), full3),
            pl.BlockSpec((D_IN, D_IN), full), pl.BlockSpec((D_IN, D_IN), full),
            pl.BlockSpec((D_IN, D_IN), full),
            pl.BlockSpec((D_IN, 2), full), pl.BlockSpec((D_IN, 2), full),
            pl.BlockSpec((1, 2), full),
            pl.BlockSpec((D_IN, D_IN), full), pl.BlockSpec((1, 8), full),
        ],
        out_specs=[
            pl.BlockSpec((BT, D_IN), blk), pl.BlockSpec((BT, D_IN), blk),
            pl.BlockSpec((BT, D_IN), blk), pl.BlockSpec((BT, D_IN), blk),
            pl.BlockSpec((BT, 2), blk),
        ],
        out_shape=[
            jax.ShapeDtypeStruct((N_TOK, D_IN), F32),
            jax.ShapeDtypeStruct((N_TOK, D_IN), F32),
            jax.ShapeDtypeStruct((N_TOK, D_IN), F32),
            jax.ShapeDtypeStruct((N_TOK, D_IN), F32),
            jax.ShapeDtypeStruct((N_TOK, 2), F32),
        ],
    )(x, h, g2, aux_upT, aux_dnT, aux_g, aux_b,
      refl_upT, refl_dnT, refl_oT, ver_upT, ver_dnT,
      vqT, vkT, vvT, wg_hT, wg_vT, vg_b, corr_oT, misc)
    acc2, vq, vk, vv, vg = t2_outs

    ones2 = jnp.ones((N_TOK, 2), F32)
    col_c = _attention(cq, ck, cv, col_oT, ones2, m[5])
    ver_c = _attention(vq, vk, vv, ver_oT, vg, m[5])

    flat_e = jnp.where(tv > 0, ti, N_EXP).reshape(-1).astype(I32)
    perm = jnp.argsort(flat_e, stable=True).astype(I32)
    sorted_e = flat_e[perm]
    counts13 = jnp.bincount(flat_e, length=N_EXP + 1).astype(I32)
    nb = (counts13[:N_EXP] + BT - 1) // BT
    bo = jnp.concatenate([jnp.zeros(1, I32), jnp.cumsum(nb)])[:N_EXP]
    gs0 = jnp.concatenate([jnp.zeros(1, I32),
                           jnp.cumsum(counts13)])[:N_EXP + 1]
    rank = jnp.arange(flat_e.shape[0], dtype=I32) - gs0[sorted_e]
    packed = jnp.where(sorted_e < N_EXP, bo[sorted_e] * BT + rank,
                       S_CAP - 1).astype(I32)
    tok_packed = jnp.zeros((S_CAP,), I32).at[packed].set(perm // K_TOP)
    w_perm = tv.reshape(-1)[perm] * m[3]
    w_packed = jnp.zeros((S_CAP,), F32).at[packed].set(w_perm)
    w_packed = w_packed.reshape(S_CAP, 1)
    pos_flat = jnp.zeros((flat_e.shape[0],), I32).at[perm].set(packed)

    tok2d = tok_packed.reshape(S_CAP, 1)
    ys = jnp.zeros((S_CAP, D_IN), F32)
    for e in range(N_EXP):
        sarr = jnp.stack([bo[e], nb[e]]).astype(I32)
        upT = p['experts_up'][e].T
        dnT = p['experts_down'][e].T
        g2d = p['exp_g'][e].reshape(1, -1)
        b2d = p['exp_b'][e].reshape(1, -1)
        ys = _expert_call(e, sarr, x, tok2d, upT, dnT, g2d, b2d,
                          w_packed, ys)
    moe = _sc_combine(ys, pos_flat)

    return acc1 + acc2 + col_c + ver_c + moe
